# baseline (speedup 1.0000x reference)
import jax
import jax.numpy as jnp
from jax.experimental import pallas as pl
from jax.experimental.pallas import tpu as pltpu

D = 128
L = 4
C_IN = 22

_BF = jnp.bfloat16
_F32 = jnp.float32


def _dot(a, b):
    return jax.lax.dot_general(a, b, (((1,), (0,)), ((), ())),
                               preferred_element_type=_F32)


def _dot_nt(a, b):
    return jax.lax.dot_general(a, b, (((1,), (1,)), ((), ())),
                               preferred_element_type=_F32)


def _mmd(x, w_bf):
    return _dot(x.astype(_BF), w_bf)


def _mmd_nt(w_bf, x):
    return _dot_nt(w_bf, x.astype(_BF))


def _lsum(x):
    acc = x[:, 0:8]
    for k in range(1, 16):
        acc = acc + x[:, 8 * k:8 * k + 8]
    w = 8
    while w > 1:
        acc = acc[:, :w // 2] + acc[:, w // 2:w]
        w //= 2
    return acc


def _ln(x, g, b):
    m = _lsum(x) / 128.0
    v = _lsum((x - m) ** 2) / 128.0
    return (x - m) / jnp.sqrt(v + 1e-5) * g + b


def _fwd_kernel(x_ref, w1T, b1, w2T, b2, demb, semb,
                vcT, vcB, vsT, vsB, nsG, nsB, ncG, ncB,
                ff1T, ff1B, ff2T, ff2B, nfG, nfB,
                pi1T, pi1B, pi2W, pi2B, v1T, v1B, v2W, v2B,
                ur1T, ur1B, ur2W, ur2B, r1T, r1B, r2W, r2B,
                pol_ref, rank_ref, val_ref, ur_ref, s729, s243):
    relu = jax.nn.relu
    xv = x_ref[0]
    h1 = relu(_mmd(xv, w1T[...]) + b1[...])
    cell = _mmd(h1, w2T[...]) + b2[...]
    emb = demb[...]
    for i in range(81):
        s729[9 * i:9 * i + 9] = jnp.broadcast_to(cell[i:i + 1], (9, D)) + emb
    cand = s729[...]
    sf = semb[...]
    for l in range(L):
        msgs = _mmd(cand, vcT[l]) + vcB[l]
        s729[...] = msgs
        for r in range(9):
            acc = s729[81 * r:81 * r + 9]
            for c in range(1, 9):
                acc = acc + s729[81 * r + 9 * c:81 * r + 9 * c + 9]
            s243[9 * r:9 * r + 9] = acc
        for c in range(9):
            acc = s729[9 * c:9 * c + 9]
            for r in range(1, 9):
                acc = acc + s729[81 * r + 9 * c:81 * r + 9 * c + 9]
            s243[81 + 9 * c:81 + 9 * c + 9] = acc
        for bb in range(9):
            Rb, Cb = divmod(bb, 3)
            acc = None
            for rr in range(3):
                for cc in range(3):
                    o = 81 * (3 * Rb + rr) + 9 * (3 * Cb + cc)
                    blk = s729[o:o + 9]
                    acc = blk if acc is None else acc + blk
            s243[162 + 9 * bb:162 + 9 * bb + 9] = acc
        sf = _ln(sf + s243[...] / 9.0, nsG[l], nsB[l])
        s243[...] = sf
        for r in range(9):
            sfr = s243[9 * r:9 * r + 9]
            for c in range(9):
                bb = (r // 3) * 3 + c // 3
                blk = (sfr + s243[162 + 9 * bb:162 + 9 * bb + 9]) \
                    + s243[81 + 9 * c:81 + 9 * c + 9]
                s729[81 * r + 9 * c:81 * r + 9 * c + 9] = blk
        gath = s729[...] / 3.0
        cu = _mmd(gath, vsT[l]) + vsB[l]
        cand = _ln(cand + cu, ncG[l], ncB[l])
        hff = relu(_mmd(cand, ff1T[l]) + ff1B[l])
        ffo = _mmd(hff, ff2T[l]) + ff2B[l]
        cand = _ln(cand + ffo, nfG[l], nfB[l])
    hpi = relu(_mmd(cand, pi1T[...]) + pi1B[...])
    pol_ref[0] = _mmd_nt(pi2W[...], hpi) + pi2B[...]
    hr = relu(_mmd(cand, r1T[...]) + r1B[...])
    rank_ref[0] = jax.nn.sigmoid(_mmd_nt(r2W[...], hr) + r2B[...])
    gmax = jnp.max(cand, axis=0, keepdims=True)
    hv = relu(_mmd(gmax, v1T[...]) + v1B[...])
    val = jnp.tanh(_mmd_nt(v2W[...], hv) + v2B[...])
    val_ref[0] = jnp.broadcast_to(val, (1, 128))
    hu = relu(_mmd(gmax, ur1T[...]) + ur1B[...])
    urv = jax.nn.sigmoid(_mmd_nt(ur2W[...], hu) + ur2B[...])
    ur_ref[0] = jnp.broadcast_to(urv, (1, 128))


def kernel(x, params):
    p = params
    B = x.shape[0]
    x2 = x.reshape(B, C_IN, 81).transpose(0, 2, 1)
    lys = p['layers']

    def stkw(name):
        return jnp.stack([lp[name].T.astype(_BF) for lp in lys])

    def stkb(name):
        return jnp.stack([lp[name][None, :] for lp in lys])

    operands = [
        x2,
        p['conv1_w'].T.astype(_BF), p['conv1_b'][None, :],
        p['conv2_w'].T.astype(_BF), p['conv2_b'][None, :],
        p['digit_embed'], p['set_embed'],
        stkw('vc_w'), stkb('vc_b'),
        stkw('vs_w'), stkb('vs_b'),
        stkb('ns_g'), stkb('ns_b'),
        stkb('nc_g'), stkb('nc_b'),
        stkw('ff1_w'), stkb('ff1_b'),
        stkw('ff2_w'), stkb('ff2_b'),
        stkb('nf_g'), stkb('nf_b'),
        p['pi1_w'].T.astype(_BF), p['pi1_b'][None, :],
        p['pi2_w'].astype(_BF), p['pi2_b'][:, None],
        p['v1_w'].T.astype(_BF), p['v1_b'][None, :],
        p['v2_w'].astype(_BF), p['v2_b'][:, None],
        p['ur1_w'].T.astype(_BF), p['ur1_b'][None, :],
        p['ur2_w'].astype(_BF), p['ur2_b'][:, None],
        p['r1_w'].T.astype(_BF), p['r1_b'][None, :],
        p['r2_w'].astype(_BF), p['r2_b'][:, None],
    ]

    def const_spec(arr):
        nd = arr.ndim
        return pl.BlockSpec(arr.shape, lambda i, _n=nd: (0,) * _n)

    in_specs = [pl.BlockSpec((1, 81, C_IN), lambda i: (i, 0, 0))]
    in_specs += [const_spec(a) for a in operands[1:]]

    out_shapes = [
        jax.ShapeDtypeStruct((B, 2, 729), _F32),
        jax.ShapeDtypeStruct((B, 1, 729), _F32),
        jax.ShapeDtypeStruct((B, 1, 128), _F32),
        jax.ShapeDtypeStruct((B, 1, 128), _F32),
    ]
    out_specs = [
        pl.BlockSpec((1, 2, 729), lambda i: (i, 0, 0)),
        pl.BlockSpec((1, 1, 729), lambda i: (i, 0, 0)),
        pl.BlockSpec((1, 1, 128), lambda i: (i, 0, 0)),
        pl.BlockSpec((1, 1, 128), lambda i: (i, 0, 0)),
    ]

    pol, rank, val, ur = pl.pallas_call(
        _fwd_kernel,
        grid=(B,),
        in_specs=in_specs,
        out_specs=out_specs,
        out_shape=out_shapes,
        scratch_shapes=[
            pltpu.VMEM((729, D), _F32),
            pltpu.VMEM((243, D), _F32),
        ],
        compiler_params=pltpu.CompilerParams(
            dimension_semantics=("parallel",),
        ),
    )(*operands)

    policy = pol.reshape(B, 1458)
    return (policy, val[:, 0, 0], ur[:, 0, 0], rank.reshape(B, 729))


# speedup vs baseline: 1.0011x; 1.0011x over previous
import jax
import jax.numpy as jnp
from jax.experimental import pallas as pl
from jax.experimental.pallas import tpu as pltpu

D = 128
L = 4
C_IN = 22

_BF = jnp.bfloat16
_F32 = jnp.float32


def _dot(a, b):
    return jax.lax.dot_general(a, b, (((1,), (0,)), ((), ())),
                               preferred_element_type=_F32)


def _dot_nt(a, b):
    return jax.lax.dot_general(a, b, (((1,), (1,)), ((), ())),
                               preferred_element_type=_F32)


def _mmd(x, w_bf):
    return _dot(x.astype(_BF), w_bf)


def _mmd_nt(w_bf, x):
    return _dot_nt(w_bf, x.astype(_BF))


def _lsum(x):
    acc = x
    for k in range(1, 16):
        acc = acc + pltpu.roll(x, 128 - 8 * k, 1)
    for s in (4, 2, 1):
        acc = acc + pltpu.roll(acc, 128 - s, 1)
    return acc[:, 0:1]


def _ln(x, g, b):
    m = _lsum(x) / 128.0
    v = _lsum((x - m) ** 2) / 128.0
    return (x - m) / jnp.sqrt(v + 1e-5) * g + b


def _fwd_kernel(x_ref, w1T, b1, w2T, b2, demb, semb,
                vcT, vcB, vsT, vsB, nsG, nsB, ncG, ncB,
                ff1T, ff1B, ff2T, ff2B, nfG, nfB,
                pi1T, pi1B, pi2W, pi2B, v1T, v1B, v2W, v2B,
                ur1T, ur1B, ur2W, ur2B, r1T, r1B, r2W, r2B,
                pol_ref, rank_ref, val_ref, ur_ref, s729, s243):
    relu = jax.nn.relu
    xv = x_ref[0]
    h1 = relu(_mmd(xv, w1T[...]) + b1[...])
    cell = _mmd(h1, w2T[...]) + b2[...]
    emb = demb[...]
    for i in range(81):
        s729[9 * i:9 * i + 9] = jnp.broadcast_to(cell[i:i + 1], (9, D)) + emb
    cand = s729[...]
    sf = semb[...]
    for l in range(L):
        msgs = _mmd(cand, vcT[l]) + vcB[l]
        s729[...] = msgs
        for r in range(9):
            acc = s729[81 * r:81 * r + 9]
            for c in range(1, 9):
                acc = acc + s729[81 * r + 9 * c:81 * r + 9 * c + 9]
            s243[9 * r:9 * r + 9] = acc
        for c in range(9):
            acc = s729[9 * c:9 * c + 9]
            for r in range(1, 9):
                acc = acc + s729[81 * r + 9 * c:81 * r + 9 * c + 9]
            s243[81 + 9 * c:81 + 9 * c + 9] = acc
        for bb in range(9):
            Rb, Cb = divmod(bb, 3)
            acc = None
            for rr in range(3):
                for cc in range(3):
                    o = 81 * (3 * Rb + rr) + 9 * (3 * Cb + cc)
                    blk = s729[o:o + 9]
                    acc = blk if acc is None else acc + blk
            s243[162 + 9 * bb:162 + 9 * bb + 9] = acc
        sf = _ln(sf + s243[...] / 9.0, nsG[l], nsB[l])
        s243[...] = sf
        for r in range(9):
            sfr = s243[9 * r:9 * r + 9]
            for c in range(9):
                bb = (r // 3) * 3 + c // 3
                blk = (sfr + s243[162 + 9 * bb:162 + 9 * bb + 9]) \
                    + s243[81 + 9 * c:81 + 9 * c + 9]
                s729[81 * r + 9 * c:81 * r + 9 * c + 9] = blk
        gath = s729[...] / 3.0
        cu = _mmd(gath, vsT[l]) + vsB[l]
        cand = _ln(cand + cu, ncG[l], ncB[l])
        hff = relu(_mmd(cand, ff1T[l]) + ff1B[l])
        ffo = _mmd(hff, ff2T[l]) + ff2B[l]
        cand = _ln(cand + ffo, nfG[l], nfB[l])
    hpi = relu(_mmd(cand, pi1T[...]) + pi1B[...])
    pol_ref[0] = _mmd_nt(pi2W[...], hpi) + pi2B[...]
    hr = relu(_mmd(cand, r1T[...]) + r1B[...])
    rank_ref[0] = jax.nn.sigmoid(_mmd_nt(r2W[...], hr) + r2B[...])
    gmax = jnp.max(cand, axis=0, keepdims=True)
    hv = relu(_mmd(gmax, v1T[...]) + v1B[...])
    val = jnp.tanh(_mmd_nt(v2W[...], hv) + v2B[...])
    val_ref[0] = jnp.broadcast_to(val, (1, 128))
    hu = relu(_mmd(gmax, ur1T[...]) + ur1B[...])
    urv = jax.nn.sigmoid(_mmd_nt(ur2W[...], hu) + ur2B[...])
    ur_ref[0] = jnp.broadcast_to(urv, (1, 128))


def kernel(x, params):
    p = params
    B = x.shape[0]
    x2 = x.reshape(B, C_IN, 81).transpose(0, 2, 1)
    lys = p['layers']

    def stkw(name):
        return jnp.stack([lp[name].T.astype(_BF) for lp in lys])

    def stkb(name):
        return jnp.stack([lp[name][None, :] for lp in lys])

    operands = [
        x2,
        p['conv1_w'].T.astype(_BF), p['conv1_b'][None, :],
        p['conv2_w'].T.astype(_BF), p['conv2_b'][None, :],
        p['digit_embed'], p['set_embed'],
        stkw('vc_w'), stkb('vc_b'),
        stkw('vs_w'), stkb('vs_b'),
        stkb('ns_g'), stkb('ns_b'),
        stkb('nc_g'), stkb('nc_b'),
        stkw('ff1_w'), stkb('ff1_b'),
        stkw('ff2_w'), stkb('ff2_b'),
        stkb('nf_g'), stkb('nf_b'),
        p['pi1_w'].T.astype(_BF), p['pi1_b'][None, :],
        p['pi2_w'].astype(_BF), p['pi2_b'][:, None],
        p['v1_w'].T.astype(_BF), p['v1_b'][None, :],
        p['v2_w'].astype(_BF), p['v2_b'][:, None],
        p['ur1_w'].T.astype(_BF), p['ur1_b'][None, :],
        p['ur2_w'].astype(_BF), p['ur2_b'][:, None],
        p['r1_w'].T.astype(_BF), p['r1_b'][None, :],
        p['r2_w'].astype(_BF), p['r2_b'][:, None],
    ]

    def const_spec(arr):
        nd = arr.ndim
        return pl.BlockSpec(arr.shape, lambda i, _n=nd: (0,) * _n)

    in_specs = [pl.BlockSpec((1, 81, C_IN), lambda i: (i, 0, 0))]
    in_specs += [const_spec(a) for a in operands[1:]]

    out_shapes = [
        jax.ShapeDtypeStruct((B, 2, 729), _F32),
        jax.ShapeDtypeStruct((B, 1, 729), _F32),
        jax.ShapeDtypeStruct((B, 1, 128), _F32),
        jax.ShapeDtypeStruct((B, 1, 128), _F32),
    ]
    out_specs = [
        pl.BlockSpec((1, 2, 729), lambda i: (i, 0, 0)),
        pl.BlockSpec((1, 1, 729), lambda i: (i, 0, 0)),
        pl.BlockSpec((1, 1, 128), lambda i: (i, 0, 0)),
        pl.BlockSpec((1, 1, 128), lambda i: (i, 0, 0)),
    ]

    pol, rank, val, ur = pl.pallas_call(
        _fwd_kernel,
        grid=(B,),
        in_specs=in_specs,
        out_specs=out_specs,
        out_shape=out_shapes,
        scratch_shapes=[
            pltpu.VMEM((729, D), _F32),
            pltpu.VMEM((243, D), _F32),
        ],
        compiler_params=pltpu.CompilerParams(
            dimension_semantics=("parallel",),
        ),
    )(*operands)

    policy = pol.reshape(B, 1458)
    return (policy, val[:, 0, 0], ur[:, 0, 0], rank.reshape(B, 729))
